# Initial kernel scaffold; baseline (speedup 1.0000x reference)
import jax
import jax.numpy as jnp
from jax.experimental import pallas as pl


def kernel(x, emb_table, W, b, a_l, a_r, bn_gamma, bn_beta, fc_w, fc_b):
    raise NotImplementedError("write your pallas kernel here")


# speedup vs baseline: 14.2566x; 14.2566x over previous
import jax
import jax.numpy as jnp
from jax.experimental import pallas as pl
from jax.experimental.pallas import tpu as pltpu

B, N, T, D, K = 8, 2048, 5, 64, 21
RB = 256
NI = N // RB
CNT = float(B * N)


def _main_body(xt_ref, xtb_ref, emb_ref, embb_ref, nrow_ref, ncol_ref,
               W_ref, bv_ref, al_ref, ar_ref, out_ref, sums_ref):
    i = pl.program_id(0)

    @pl.when(i == 0)
    def _():
        sums_ref[...] = jnp.zeros_like(sums_ref)

    emb = emb_ref[...]
    embb = embb_ref[...]
    raw = jax.lax.dot_general(embb, emb, (((1,), (1,)), ((), ())),
                              preferred_element_type=jnp.float32)
    cos = raw / (ncol_ref[...] * nrow_ref[...])

    work = cos
    for _ in range(K - 1):
        m = jnp.max(work, axis=1, keepdims=True)
        work = jnp.where(work == m, jnp.float32(-1e30), work)
    thr = jnp.max(work, axis=1, keepdims=True)
    mask = cos >= thr

    W = W_ref[...]
    bv = bv_ref[...]
    al1, al2 = al_ref[0:1, :], al_ref[1:2, :]
    ar1, ar2 = ar_ref[0:1, :], ar_ref[1:2, :]

    er_emb = jax.lax.dot_general(ar2, emb, (((1,), (1,)), ((), ())),
                                 preferred_element_type=jnp.float32)
    el_embb = jnp.sum(embb * al2, axis=1, keepdims=True)

    s1 = jnp.zeros((1, D), jnp.float32)
    s2 = jnp.zeros((1, D), jnp.float32)
    for bb in range(B):
        xb = xt_ref[bb]
        g = jax.lax.dot_general(xb, W, (((0,), (1,)), ((), ())),
                                preferred_element_type=jnp.float32) + bv
        xbb = xtb_ref[bb]
        gb = jax.lax.dot_general(xbb, W, (((0,), (1,)), ((), ())),
                                 preferred_element_type=jnp.float32) + bv
        el = jnp.sum(gb * al1, axis=1, keepdims=True) + el_embb
        er = jax.lax.dot_general(ar1, g, (((1,), (1,)), ((), ())),
                                 preferred_element_type=jnp.float32) + er_emb
        e = el + er
        e = jnp.where(e >= 0, e, 0.2 * e)
        e = jnp.where(mask, e, jnp.float32(-1e9))
        m2 = jnp.max(e, axis=1, keepdims=True)
        w = jnp.exp(e - m2)
        w = jnp.where(mask, w, jnp.float32(0.0))
        dn = jnp.sum(w, axis=1, keepdims=True)
        z = jax.lax.dot_general(w, g, (((1,), (0,)), ((), ())),
                                preferred_element_type=jnp.float32)
        z = jnp.maximum(z / dn, 0.0)
        op = z * embb
        out_ref[bb] = op
        s1 = s1 + jnp.sum(op, axis=0, keepdims=True)
        s2 = s2 + jnp.sum(op * op, axis=0, keepdims=True)
    sums_ref[0:1, :] += s1
    sums_ref[1:2, :] += s2


def _bnfc_body(op_ref, sums_ref, gam_ref, bet_ref, fw_ref, fb_ref, y_ref):
    mean = sums_ref[0:1, :] * (1.0 / CNT)
    var = sums_ref[1:2, :] * (1.0 / CNT) - mean * mean
    scale = jax.lax.rsqrt(var + 1e-5) * gam_ref[...]
    shift = bet_ref[...] - mean * scale
    fw = fw_ref[...]
    fb = fb_ref[...]
    for bb in range(B):
        o = op_ref[bb] * scale + shift
        o = jnp.maximum(o, 0.0)
        r = jax.lax.dot_general(fw, o, (((1,), (1,)), ((), ())),
                                preferred_element_type=jnp.float32)
        y_ref[bb:bb + 1, :] = r + fb


def kernel(x, emb_table, W, b, a_l, a_r, bn_gamma, bn_beta, fc_w, fc_b):
    xt = jnp.transpose(x, (0, 2, 1))
    bv = b.reshape(1, D)
    al = a_l.reshape(2, D)
    ar = a_r.reshape(2, D)
    nn = jnp.linalg.norm(emb_table, axis=-1)
    nrow = nn.reshape(1, N)
    ncol = nn.reshape(N, 1)

    out_pre, sums = pl.pallas_call(
        _main_body,
        grid=(NI,),
        in_specs=[
            pl.BlockSpec((B, T, N), lambda i: (0, 0, 0)),
            pl.BlockSpec((B, T, RB), lambda i: (0, 0, i)),
            pl.BlockSpec((N, D), lambda i: (0, 0)),
            pl.BlockSpec((RB, D), lambda i: (i, 0)),
            pl.BlockSpec((1, N), lambda i: (0, 0)),
            pl.BlockSpec((RB, 1), lambda i: (i, 0)),
            pl.BlockSpec((D, T), lambda i: (0, 0)),
            pl.BlockSpec((1, D), lambda i: (0, 0)),
            pl.BlockSpec((2, D), lambda i: (0, 0)),
            pl.BlockSpec((2, D), lambda i: (0, 0)),
        ],
        out_specs=[
            pl.BlockSpec((B, RB, D), lambda i: (0, i, 0)),
            pl.BlockSpec((2, D), lambda i: (0, 0)),
        ],
        out_shape=[
            jax.ShapeDtypeStruct((B, N, D), jnp.float32),
            jax.ShapeDtypeStruct((2, D), jnp.float32),
        ],
        compiler_params=pltpu.CompilerParams(
            dimension_semantics=("arbitrary",)),
    )(xt, xt, emb_table, emb_table, nrow, ncol, W, bv, al, ar)

    y = pl.pallas_call(
        _bnfc_body,
        grid=(NI,),
        in_specs=[
            pl.BlockSpec((B, RB, D), lambda i: (0, i, 0)),
            pl.BlockSpec((2, D), lambda i: (0, 0)),
            pl.BlockSpec((1, D), lambda i: (0, 0)),
            pl.BlockSpec((1, D), lambda i: (0, 0)),
            pl.BlockSpec((1, D), lambda i: (0, 0)),
            pl.BlockSpec((1, 1), lambda i: (0, 0)),
        ],
        out_specs=pl.BlockSpec((B, RB), lambda i: (0, i)),
        out_shape=jax.ShapeDtypeStruct((B, N), jnp.float32),
        compiler_params=pltpu.CompilerParams(
            dimension_semantics=("arbitrary",)),
    )(out_pre, sums, bn_gamma.reshape(1, D), bn_beta.reshape(1, D),
      fc_w.reshape(1, D), fc_b.reshape(1, 1))
    return y
